# baseline (speedup 1.0000x reference)
import functools

import jax
import jax.numpy as jnp
from jax import lax
from jax.experimental import pallas as pl
from jax.experimental.pallas import tpu as pltpu
from jax.experimental.pallas import tpu_sc as plsc


def _gather_body(nc, b_per_w, table_hbm, idx_hbm, out_hbm, idx_v, rows_v, sem):
    wid = lax.axis_index("s") * nc + lax.axis_index("c")
    base = wid * b_per_w
    pltpu.sync_copy(idx_hbm.at[pl.ds(base, b_per_w)], idx_v)
    pltpu.async_copy(table_hbm.at[idx_v], rows_v, sem).wait()
    pltpu.sync_copy(rows_v, out_hbm.at[pl.ds(base, b_per_w)])


def _sc_gather(embeddings, idx):
    vocab, dim = embeddings.shape
    batch = idx.shape[0]
    info = plsc.get_sparse_core_info()
    nc, ns = info.num_cores, info.num_subcores
    nw = nc * ns
    b_per_w = batch // nw
    mesh = plsc.VectorSubcoreMesh(core_axis_name="c", subcore_axis_name="s")
    k = pl.kernel(
        functools.partial(_gather_body, nc, b_per_w),
        out_type=jax.ShapeDtypeStruct((batch, dim), jnp.float32),
        mesh=mesh,
        scratch_types=[
            pltpu.VMEM((b_per_w,), jnp.int32),
            pltpu.VMEM((b_per_w, dim), jnp.float32),
            pltpu.SemaphoreType.DMA,
        ],
        compiler_params=pltpu.CompilerParams(use_tc_tiling_on_sc=False),
    )
    return k(embeddings, idx)


def _proj_body(wt_ref, e_ref, b_ref, *rest):
    o_ref = rest[-1]
    o_ref[...] = lax.dot_general(
        wt_ref[...], e_ref[...], (((0,), (1,)), ((), ())),
        preferred_element_type=jnp.float32,
    ) + jnp.transpose(b_ref[...])


def _proj_chunk(embed_chunk, Wt, b_row, batch, chunk, out_buf=None, vt=1024):
    bc, dim = embed_chunk.shape
    vocab = Wt.shape[1]
    nvt = pl.cdiv(vocab, vt)
    in_specs = [
        pl.BlockSpec((dim, vt), lambda j: (0, j)),
        pl.BlockSpec((bc, dim), lambda j: (0, 0)),
        pl.BlockSpec((1, vt), lambda j: (0, j)),
    ]
    args = [Wt, embed_chunk, b_row]
    aliases = {}
    if out_buf is not None:
        in_specs.append(pl.BlockSpec(memory_space=pl.ANY))
        args.append(out_buf)
        aliases = {3: 0}
    return pl.pallas_call(
        _proj_body,
        grid=(nvt,),
        in_specs=in_specs,
        out_specs=pl.BlockSpec((vt, bc), lambda j: (j, chunk)),
        out_shape=jax.ShapeDtypeStruct((vocab, batch), jnp.float32),
        input_output_aliases=aliases,
        compiler_params=pltpu.CompilerParams(
            dimension_semantics=("arbitrary",),
            vmem_limit_bytes=48 * 1024 * 1024,
        ),
    )(*args)


def kernel(target_word_idx, embeddings, W, b):
    idx = target_word_idx.astype(jnp.int32)
    batch = idx.shape[0]
    vocab = W.shape[0]
    half = batch // 2
    e0 = _sc_gather(embeddings, idx[:half])
    e1 = _sc_gather(embeddings, idx[half:])
    Wt = W.T
    b_row = b.reshape(1, -1)
    out_buf = _proj_chunk(e0, Wt, b_row, batch, 0)
    out_buf = _proj_chunk(e1, Wt, b_row, batch, 1, out_buf=out_buf)
    return out_buf.T


# speedup vs baseline: 1.0380x; 1.0380x over previous
import functools

import jax
import jax.numpy as jnp
from jax import lax
from jax.experimental import pallas as pl
from jax.experimental.pallas import tpu as pltpu
from jax.experimental.pallas import tpu_sc as plsc


def _gather_body(nc, b_per_w, table_hbm, idx_hbm, out_hbm, idx_v, rows_v, sem):
    wid = lax.axis_index("s") * nc + lax.axis_index("c")
    base = wid * b_per_w
    pltpu.sync_copy(idx_hbm.at[pl.ds(base, b_per_w)], idx_v)
    pltpu.async_copy(table_hbm.at[idx_v], rows_v, sem).wait()
    pltpu.sync_copy(rows_v, out_hbm.at[pl.ds(base, b_per_w)])


def _sc_gather(embeddings, idx):
    vocab, dim = embeddings.shape
    batch = idx.shape[0]
    info = plsc.get_sparse_core_info()
    nc, ns = info.num_cores, info.num_subcores
    nw = nc * ns
    b_per_w = batch // nw
    mesh = plsc.VectorSubcoreMesh(core_axis_name="c", subcore_axis_name="s")
    k = pl.kernel(
        functools.partial(_gather_body, nc, b_per_w),
        out_type=jax.ShapeDtypeStruct((batch, dim), jnp.float32),
        mesh=mesh,
        scratch_types=[
            pltpu.VMEM((b_per_w,), jnp.int32),
            pltpu.VMEM((b_per_w, dim), jnp.float32),
            pltpu.SemaphoreType.DMA,
        ],
        compiler_params=pltpu.CompilerParams(use_tc_tiling_on_sc=False),
    )
    return k(embeddings, idx)


def _proj_body(wt_ref, e_ref, b_ref, o_ref):
    o_ref[...] = lax.dot_general(
        wt_ref[...], e_ref[...], (((0,), (1,)), ((), ())),
        preferred_element_type=jnp.float32,
    ) + jnp.transpose(b_ref[...])


def _tc_project_t(embed, Wt, b_row, vt=1024):
    batch, dim = embed.shape
    vocab = Wt.shape[1]
    nvt = pl.cdiv(vocab, vt)
    return pl.pallas_call(
        _proj_body,
        grid=(nvt,),
        in_specs=[
            pl.BlockSpec((dim, vt), lambda j: (0, j)),
            pl.BlockSpec((batch, dim), lambda j: (0, 0)),
            pl.BlockSpec((1, vt), lambda j: (0, j)),
        ],
        out_specs=pl.BlockSpec((vt, batch), lambda j: (j, 0)),
        out_shape=jax.ShapeDtypeStruct((vocab, batch), jnp.float32),
        compiler_params=pltpu.CompilerParams(
            dimension_semantics=("arbitrary",),
        ),
    )(Wt, embed, b_row)


def kernel(target_word_idx, embeddings, W, b):
    idx = target_word_idx.astype(jnp.int32)
    embed = _sc_gather(embeddings, idx)
    out_t = _tc_project_t(embed, W.T, b.reshape(1, -1))
    return out_t.T
